# Initial kernel scaffold; baseline (speedup 1.0000x reference)
import jax
import jax.numpy as jnp
from jax.experimental import pallas as pl


def kernel(y, embed_table, W, b):
    raise NotImplementedError("write your pallas kernel here")


# speedup vs baseline: 3.9055x; 3.9055x over previous
import jax
import jax.numpy as jnp
from jax import lax
from jax.experimental import pallas as pl
from jax.experimental.pallas import tpu as pltpu
from jax.experimental.pallas import tpu_sc as plsc

_B, _L, _D = 16384, 200, 10
_M = _B * _L
_DP = 16
_NC, _NS = 2, 16
_NW = _NC * _NS
_PER_W = _M // _NW
_CH = 2048
_NCHUNK = _PER_W // _CH


def _body(y_hbm, emb_hbm, w_hbm, b_hbm, out_hbm, yv, outv, lut, embv, wv, bv):
    pltpu.sync_copy(emb_hbm, embv)
    pltpu.sync_copy(w_hbm, wv)
    pltpu.sync_copy(b_hbm, bv)
    for i in range(10):
        acc = bv[...]
        for k in range(10):
            e = embv[pl.ds((i * 10 + k) * _DP, _DP)]
            acc = acc + e * wv[pl.ds(k * _DP, _DP)]
        lut[pl.ds(i * _DP, _DP)] = jnp.maximum(acc, 0.0)

    wid = lax.axis_index("s") * _NC + lax.axis_index("c")
    base = wid * _PER_W
    iota = lax.iota(jnp.int32, 16)
    pos = [iota + p * 16 for p in range(5)]
    yoff = [p_ // 10 for p_ in pos]
    fpat = [p_ % 10 for p_ in pos]

    def chunk(c, carry):
        off = base + c * _CH
        pltpu.sync_copy(y_hbm.at[pl.ds(off, _CH)], yv)

        def group(g, carry2):
            ib = jnp.full((16,), g * 8, jnp.int32)
            for p in range(5):
                rows = plsc.load_gather(yv, [ib + yoff[p]])
                vals = plsc.load_gather(lut, [rows * _DP + fpat[p]])
                outv[pl.ds(g * 80 + p * 16, 16)] = vals
            return carry2

        lax.fori_loop(0, _CH // 8, group, 0)
        pltpu.sync_copy(outv, out_hbm.at[pl.ds(off * _D, _CH * _D)])
        return carry

    lax.fori_loop(0, _NCHUNK, chunk, 0)


def kernel(y, embed_table, W, b):
    y_flat = y.reshape(-1).astype(jnp.int32)
    embp = jnp.repeat(embed_table.reshape(-1), _DP)
    wp = jnp.pad(W, ((0, 0), (0, _DP - _D))).reshape(-1)
    bp = jnp.pad(b, (0, _DP - _D))
    mesh = plsc.VectorSubcoreMesh(core_axis_name="c", subcore_axis_name="s")
    out_flat = pl.kernel(
        _body,
        out_type=jax.ShapeDtypeStruct((_M * _D,), jnp.float32),
        mesh=mesh,
        compiler_params=pltpu.CompilerParams(needs_layout_passes=False),
        scratch_types=[
            pltpu.VMEM((_CH,), jnp.int32),
            pltpu.VMEM((_CH * _D,), jnp.float32),
            pltpu.VMEM((10 * _DP,), jnp.float32),
            pltpu.VMEM((100 * _DP,), jnp.float32),
            pltpu.VMEM((10 * _DP,), jnp.float32),
            pltpu.VMEM((_DP,), jnp.float32),
        ],
    )(y_flat, embp, wp, bp)
    return out_flat.reshape(_B, _L, _D)


# speedup vs baseline: 4.7720x; 1.2219x over previous
import jax
import jax.numpy as jnp
from jax import lax
from jax.experimental import pallas as pl
from jax.experimental.pallas import tpu as pltpu
from jax.experimental.pallas import tpu_sc as plsc

_B, _L, _D = 16384, 200, 10
_M = _B * _L
_DP = 16
_NC, _NS = 2, 16
_NW = _NC * _NS
_PER_W = _M // _NW
_CH = 2048
_NCHUNK = _PER_W // _CH


def _body(y_hbm, emb_hbm, w_hbm, b_hbm, out_hbm, yv, outv, lut, embv, wv, bv):
    pltpu.sync_copy(emb_hbm, embv)
    pltpu.sync_copy(w_hbm, wv)
    pltpu.sync_copy(b_hbm, bv)
    for i in range(10):
        acc = bv[...]
        for k in range(10):
            e = embv[pl.ds((i * 10 + k) * _DP, _DP)]
            acc = acc + e * wv[pl.ds(k * _DP, _DP)]
        lut[pl.ds(i * _DP, _DP)] = jnp.maximum(acc, 0.0)

    wid = lax.axis_index("s") * _NC + lax.axis_index("c")
    base = wid * _PER_W
    iota = lax.iota(jnp.int32, 16)
    pos = [iota + p * 16 for p in range(5)]
    yoff = [p_ // 10 for p_ in pos]
    fpat = [p_ % 10 for p_ in pos]

    def chunk(c, carry):
        off = base + c * _CH
        pltpu.sync_copy(y_hbm.at[pl.ds(off, _CH)], yv)

        @plsc.parallel_loop(0, _CH // 8, unroll=8)
        def _group(g):
            ib = jnp.full((16,), g * 8, jnp.int32)
            for p in range(5):
                rows = plsc.load_gather(yv, [ib + yoff[p]])
                vals = plsc.load_gather(lut, [rows * _DP + fpat[p]])
                outv[pl.ds(g * 80 + p * 16, 16)] = vals
        pltpu.sync_copy(outv, out_hbm.at[pl.ds(off * _D, _CH * _D)])
        return carry

    lax.fori_loop(0, _NCHUNK, chunk, 0)


def kernel(y, embed_table, W, b):
    y_flat = y.reshape(-1).astype(jnp.int32)
    embp = jnp.repeat(embed_table.reshape(-1), _DP)
    wp = jnp.pad(W, ((0, 0), (0, _DP - _D))).reshape(-1)
    bp = jnp.pad(b, (0, _DP - _D))
    mesh = plsc.VectorSubcoreMesh(core_axis_name="c", subcore_axis_name="s")
    out_flat = pl.kernel(
        _body,
        out_type=jax.ShapeDtypeStruct((_M * _D,), jnp.float32),
        mesh=mesh,
        compiler_params=pltpu.CompilerParams(needs_layout_passes=False),
        scratch_types=[
            pltpu.VMEM((_CH,), jnp.int32),
            pltpu.VMEM((_CH * _D,), jnp.float32),
            pltpu.VMEM((10 * _DP,), jnp.float32),
            pltpu.VMEM((100 * _DP,), jnp.float32),
            pltpu.VMEM((10 * _DP,), jnp.float32),
            pltpu.VMEM((_DP,), jnp.float32),
        ],
    )(y_flat, embp, wp, bp)
    return out_flat.reshape(_B, _L, _D)


# speedup vs baseline: 62.3133x; 13.0581x over previous
import jax
import jax.numpy as jnp
from jax import lax
from jax.experimental import pallas as pl
from jax.experimental.pallas import tpu as pltpu
from jax.experimental.pallas import tpu_sc as plsc

_B, _L, _D = 16384, 200, 10
_DP = 16
_NC, _NS = 2, 16
_NW = _NC * _NS
_BSPAN = _B // _NW
_LT = 8
_NSTRIP = _L // _LT
_NJ = _LT * _BSPAN // 16


def _body(yt_hbm, emb_hbm, w_hbm, b_hbm, q_hbm,
          ytv0, ytv1, qv0, qv1, embv, wv, bv, lut, si0, si1, so0, so1):
    wid = lax.axis_index("s") * _NC + lax.axis_index("c")
    b0 = wid * _BSPAN
    ytv, qv, si, so = [ytv0, ytv1], [qv0, qv1], [si0, si1], [so0, so1]

    in_h = [None, None]
    in_h[0] = pltpu.async_copy(
        yt_hbm.at[pl.ds(0, _LT), pl.ds(b0, _BSPAN)], ytv[0], si[0])

    pltpu.sync_copy(emb_hbm, embv)
    pltpu.sync_copy(w_hbm, wv)
    pltpu.sync_copy(b_hbm, bv)
    for i in range(10):
        acc = bv[...]
        for k in range(10):
            e = embv[pl.ds((i * 10 + k) * _DP, _DP)]
            acc = acc + e * wv[pl.ds(k * _DP, _DP)]
        lut[pl.ds(i * _DP, _DP)] = jnp.maximum(acc, 0.0)

    out_h = [[], []]
    for c in range(_NSTRIP):
        p = c % 2
        if c + 1 < _NSTRIP:
            in_h[1 - p] = pltpu.async_copy(
                yt_hbm.at[pl.ds((c + 1) * _LT, _LT), pl.ds(b0, _BSPAN)],
                ytv[1 - p], si[1 - p])
        in_h[p].wait()
        for h in out_h[p]:
            h.wait()
        out_h[p] = []

        ytv_p, qv_p = ytv[p], qv[p]

        @plsc.parallel_loop(0, _NJ, unroll=2)
        def _j(j):
            jl = j // (_BSPAN // 16)
            jb = (j % (_BSPAN // 16)) * 16
            rows = ytv_p[jl, pl.ds(jb, 16)]
            rb = rows * _DP
            for f in range(10):
                qv_p[f, jl, pl.ds(jb, 16)] = plsc.load_gather(lut, [rb + f])

        for f in range(10):
            out_h[p].append(pltpu.async_copy(
                qv_p.at[f],
                q_hbm.at[f, pl.ds(c * _LT, _LT), pl.ds(b0, _BSPAN)], so[p]))
    for p in (0, 1):
        for h in out_h[p]:
            h.wait()


def kernel(y, embed_table, W, b):
    yt = y.T.astype(jnp.int32)
    embp = jnp.repeat(embed_table.reshape(-1), _DP)
    wp = jnp.pad(W, ((0, 0), (0, _DP - _D))).reshape(-1)
    bp = jnp.pad(b, (0, _DP - _D))
    mesh = plsc.VectorSubcoreMesh(core_axis_name="c", subcore_axis_name="s")
    q = pl.kernel(
        _body,
        out_type=jax.ShapeDtypeStruct((_D, _L, _B), jnp.float32),
        mesh=mesh,
        compiler_params=pltpu.CompilerParams(needs_layout_passes=False),
        scratch_types=[
            pltpu.VMEM((_LT, _BSPAN), jnp.int32),
            pltpu.VMEM((_LT, _BSPAN), jnp.int32),
            pltpu.VMEM((_D, _LT, _BSPAN), jnp.float32),
            pltpu.VMEM((_D, _LT, _BSPAN), jnp.float32),
            pltpu.VMEM((100 * _DP,), jnp.float32),
            pltpu.VMEM((10 * _DP,), jnp.float32),
            pltpu.VMEM((_DP,), jnp.float32),
            pltpu.VMEM((10 * _DP,), jnp.float32),
            pltpu.SemaphoreType.DMA,
            pltpu.SemaphoreType.DMA,
            pltpu.SemaphoreType.DMA,
            pltpu.SemaphoreType.DMA,
        ],
    )(yt, embp, wp, bp)
    return q.transpose(2, 1, 0)
